# Initial kernel scaffold; baseline (speedup 1.0000x reference)
import jax
import jax.numpy as jnp
from jax.experimental import pallas as pl


def kernel(x, stem_w, stem_shift, stem_a, stem_b, s1b0_w1, s1b0_sh1, s1b0_a1, s1b0_b1, s1b0_w2, s1b0_sh2, s1b0_a2, s1b0_b2, s1b1_w1, s1b1_sh1, s1b1_a1, s1b1_b1, s1b1_w2, s1b1_sh2, s1b1_a2, s1b1_b2, s1b2_w1, s1b2_sh1, s1b2_a1, s1b2_b1, s1b2_w2, s1b2_sh2, s1b2_a2, s1b2_b2, s2h_w1, s2h_sh1, s2h_a1, s2h_b1, s2h_w2, s2h_sh2, s2h_a2, s2h_b2, s2h_ws, s2h_shs, s2b0_w1, s2b0_sh1, s2b0_a1, s2b0_b1, s2b0_w2, s2b0_sh2, s2b0_a2, s2b0_b2, s2b1_w1, s2b1_sh1, s2b1_a1, s2b1_b1, s2b1_w2, s2b1_sh2, s2b1_a2, s2b1_b2, s3h_w1, s3h_sh1, s3h_a1, s3h_b1, s3h_w2, s3h_sh2, s3h_a2, s3h_b2, s3h_ws, s3h_shs, s3b0_w1, s3b0_sh1, s3b0_a1, s3b0_b1, s3b0_w2, s3b0_sh2, s3b0_a2, s3b0_b2, s3b1_w1, s3b1_sh1, s3b1_a1, s3b1_b1, s3b1_w2, s3b1_sh2, s3b1_a2, s3b1_b2, fc_w, fc_b):
    raise NotImplementedError("write your pallas kernel here")


# speedup vs baseline: 4.3106x; 4.3106x over previous
import jax
import jax.numpy as jnp
from jax.experimental import pallas as pl
from jax.experimental.pallas import tpu as pltpu


def _pade(y, a, b):
    num = a[5:6, :]
    for i in (4, 3, 2, 1, 0):
        num = num * y + a[i:i + 1, :]
    ay = jnp.abs(y)
    y2 = y * y
    den = 1.0 + ay * (b[0:1, :] + b[2:3, :] * y2) + y2 * (b[1:2, :] + b[3:4, :] * y2)
    return num * pl.reciprocal(den, approx=True)


def _make_stage(*, Wp, Ho, n_blocks, mode, inv_hw=None):
    L = Ho * Wp
    OFF = Wp + 1

    def body(*refs):
        it = iter(refs)
        x_ref = next(it)
        mask_ref = next(it)
        n_head = 4 if mode == 'stem' else 10
        head = [next(it) for _ in range(n_head)]
        blocks = [[next(it) for _ in range(8)] for _ in range(n_blocks)]
        out_ref = next(it)
        cur = next(it)
        mid = next(it)

        m = mask_ref[...]
        cur[...] = jnp.zeros_like(cur)
        mid[...] = jnp.zeros_like(mid)

        def conv(tap, w_ref):
            acc = None
            for dy in range(3):
                for dx in range(3):
                    p = jnp.dot(tap(dy, dx), w_ref[dy * 3 + dx],
                                preferred_element_type=jnp.float32)
                    acc = p if acc is None else acc + p
            return acc

        def ctap(ref):
            return lambda dy, dx: ref[dy * Wp + dx: dy * Wp + dx + L, :]

        if mode == 'stem':
            w, sh, a, b = head
            y = conv(lambda dy, dx: x_ref[0, dy * Wp + dx: dy * Wp + dx + L, :], w)
            cur[OFF:OFF + L, :] = _pade(y + sh[...], a[...], b[...]) * m
        else:
            w1, sh1, a1, b1, w2, sh2, a2, b2, ws, shs = head

            def ptap(dy, dx):
                off = (dy // 2) * Wp + (dx // 2)
                return x_ref[0, (dy % 2) * 2 + (dx % 2), off: off + L, :]

            y1 = conv(ptap, w1) + sh1[...]
            mid[OFF:OFF + L, :] = _pade(y1, a1[...], b1[...]) * m
            res = jnp.dot(x_ref[0, 3, 0:L, :], ws[...],
                          preferred_element_type=jnp.float32) + shs[...]
            y2 = conv(ctap(mid), w2) + sh2[...] + res
            cur[OFF:OFF + L, :] = _pade(y2, a2[...], b2[...]) * m

        for (w1, sh1, a1, b1, w2, sh2, a2, b2) in blocks:
            y1 = conv(ctap(cur), w1) + sh1[...]
            mid[OFF:OFF + L, :] = _pade(y1, a1[...], b1[...]) * m
            res = cur[OFF:OFF + L, :]
            y2 = conv(ctap(mid), w2) + sh2[...] + res
            cur[OFF:OFF + L, :] = _pade(y2, a2[...], b2[...]) * m

        if mode == 'down_pool':
            out_ref[0, 0, :] = jnp.sum(cur[...], axis=0) * inv_hw
        else:
            out_ref[0, :, :] = cur[...]

    return body


def _full(arr):
    nd = arr.ndim
    return pl.BlockSpec(arr.shape, lambda n, _nd=nd: (0,) * _nd)


def _stage_call(x, mask, head, blocks, *, Wp, Ho, mode, lanes):
    N = x.shape[0]
    Lp = (Ho + 2) * Wp + 8
    args = [x, mask] + list(head) + [a for blk in blocks for a in blk]
    body = _make_stage(Wp=Wp, Ho=Ho, n_blocks=len(blocks), mode=mode,
                       inv_hw=None if mode != 'down_pool' else 1.0 / (Ho * (Wp - 2)))

    in_specs = [pl.BlockSpec((1,) + tuple(x.shape[1:]),
                             lambda n, _nd=x.ndim: (n,) + (0,) * (_nd - 1))]
    in_specs += [_full(a) for a in args[1:]]

    if mode == 'down_pool':
        out_shape = jax.ShapeDtypeStruct((N, 1, lanes), jnp.float32)
        out_spec = pl.BlockSpec((1, 1, lanes), lambda n: (n, 0, 0))
    else:
        out_shape = jax.ShapeDtypeStruct((N, Lp, lanes), jnp.float32)
        out_spec = pl.BlockSpec((1, Lp, lanes), lambda n: (n, 0, 0))

    return pl.pallas_call(
        body,
        out_shape=out_shape,
        grid=(N,),
        in_specs=in_specs,
        out_specs=out_spec,
        scratch_shapes=[pltpu.VMEM((Lp, lanes), jnp.float32),
                        pltpu.VMEM((Lp, lanes), jnp.float32)],
        compiler_params=pltpu.CompilerParams(dimension_semantics=("parallel",)),
    )(*args)


def _bd(w, g):
    eye = jnp.eye(g, dtype=w.dtype)
    if w.ndim == 2:
        ci, co = w.shape
        return jnp.einsum('ij,ab->iajb', eye, w).reshape(g * ci, g * co)
    t, ci, co = w.shape
    return jnp.einsum('ij,tab->tiajb', eye, w).reshape(t, g * ci, g * co)


def _tile(arr, g):
    return jnp.tile(arr, (1, g))


def _col_mask(ho, wp, wo):
    q = jnp.arange(ho * wp, dtype=jnp.int32)
    return ((q % wp) < wo).astype(jnp.float32)[:, None]


def _repack(o, hp, wp, g, c, wpn):
    n = o.shape[0]
    hq, wq = hp // 2, wp // 2
    t = o[:, :hp * wp, :].reshape(n, hq, 2, wq, 2, 2, g // 2, c)
    t = t.transpose(0, 5, 2, 4, 1, 3, 6, 7)
    t = t.reshape(n * 2, 4, hq, wq, (g // 2) * c)
    t = jnp.pad(t, ((0, 0), (0, 0), (0, 0), (0, wpn - wq), (0, 0)))
    t = t.reshape(n * 2, 4, hq * wpn, (g // 2) * c)
    return jnp.pad(t, ((0, 0), (0, 0), (0, 8), (0, 0)))


def kernel(x, stem_w, stem_shift, stem_a, stem_b, s1b0_w1, s1b0_sh1, s1b0_a1, s1b0_b1, s1b0_w2, s1b0_sh2, s1b0_a2, s1b0_b2, s1b1_w1, s1b1_sh1, s1b1_a1, s1b1_b1, s1b1_w2, s1b1_sh2, s1b1_a2, s1b1_b2, s1b2_w1, s1b2_sh1, s1b2_a1, s1b2_b1, s1b2_w2, s1b2_sh2, s1b2_a2, s1b2_b2, s2h_w1, s2h_sh1, s2h_a1, s2h_b1, s2h_w2, s2h_sh2, s2h_a2, s2h_b2, s2h_ws, s2h_shs, s2b0_w1, s2b0_sh1, s2b0_a1, s2b0_b1, s2b0_w2, s2b0_sh2, s2b0_a2, s2b0_b2, s2b1_w1, s2b1_sh1, s2b1_a1, s2b1_b1, s2b1_w2, s2b1_sh2, s2b1_a2, s2b1_b2, s3h_w1, s3h_sh1, s3h_a1, s3h_b1, s3h_w2, s3h_sh2, s3h_a2, s3h_b2, s3h_ws, s3h_shs, s3b0_w1, s3b0_sh1, s3b0_a1, s3b0_b1, s3b0_w2, s3b0_sh2, s3b0_a2, s3b0_b2, s3b1_w1, s3b1_sh1, s3b1_a1, s3b1_b1, s3b1_w2, s3b1_sh2, s3b1_a2, s3b1_b2, fc_w, fc_b):
    n, cin, h, w = x.shape
    g1, g2, g3 = 16, 8, 4
    n1 = n // g1
    ho1, wo1 = h, w
    ho2, wo2 = h // 2, w // 2
    ho3, wo3 = h // 4, w // 4
    wp1, wp2, wp3 = wo1 + 2, wo2 + 2, wo3 + 2

    xt = jnp.transpose(x, (0, 2, 3, 1)).astype(jnp.float32)
    xp = jnp.pad(xt, ((0, 0), (1, 1), (1, 1), (0, 8 - cin)))
    xg = xp.reshape(n1, g1, (h + 2) * (w + 2), 8).transpose(0, 2, 1, 3)
    xg = jnp.pad(xg.reshape(n1, (h + 2) * (w + 2), g1 * 8), ((0, 0), (0, 8), (0, 0)))

    def pack_blk(blk, g):
        w1, sh1, a1, b1, w2, sh2, a2, b2 = blk
        return (_bd(w1, g), _tile(sh1, g), _tile(a1, g), _tile(b1, g),
                _bd(w2, g), _tile(sh2, g), _tile(a2, g), _tile(b2, g))

    stem = (_bd(stem_w, g1), _tile(stem_shift, g1), _tile(stem_a, g1), _tile(stem_b, g1))
    s1 = [pack_blk(b, g1) for b in
          ((s1b0_w1, s1b0_sh1, s1b0_a1, s1b0_b1, s1b0_w2, s1b0_sh2, s1b0_a2, s1b0_b2),
           (s1b1_w1, s1b1_sh1, s1b1_a1, s1b1_b1, s1b1_w2, s1b1_sh2, s1b1_a2, s1b1_b2),
           (s1b2_w1, s1b2_sh1, s1b2_a1, s1b2_b1, s1b2_w2, s1b2_sh2, s1b2_a2, s1b2_b2))]
    h2 = pack_blk((s2h_w1, s2h_sh1, s2h_a1, s2h_b1, s2h_w2, s2h_sh2, s2h_a2, s2h_b2), g2) \
        + (_bd(s2h_ws, g2), _tile(s2h_shs, g2))
    s2 = [pack_blk(b, g2) for b in
          ((s2b0_w1, s2b0_sh1, s2b0_a1, s2b0_b1, s2b0_w2, s2b0_sh2, s2b0_a2, s2b0_b2),
           (s2b1_w1, s2b1_sh1, s2b1_a1, s2b1_b1, s2b1_w2, s2b1_sh2, s2b1_a2, s2b1_b2))]
    h3 = pack_blk((s3h_w1, s3h_sh1, s3h_a1, s3h_b1, s3h_w2, s3h_sh2, s3h_a2, s3h_b2), g3) \
        + (_bd(s3h_ws, g3), _tile(s3h_shs, g3))
    s3 = [pack_blk(b, g3) for b in
          ((s3b0_w1, s3b0_sh1, s3b0_a1, s3b0_b1, s3b0_w2, s3b0_sh2, s3b0_a2, s3b0_b2),
           (s3b1_w1, s3b1_sh1, s3b1_a1, s3b1_b1, s3b1_w2, s3b1_sh2, s3b1_a2, s3b1_b2))]

    o1 = _stage_call(xg, _col_mask(ho1, wp1, wo1), stem, s1,
                     Wp=wp1, Ho=ho1, mode='stem', lanes=g1 * 16)

    p2 = _repack(o1, ho1 + 2, wp1, g1, 16, wp2)
    o2 = _stage_call(p2, _col_mask(ho2, wp2, wo2), h2, s2,
                     Wp=wp2, Ho=ho2, mode='down', lanes=g2 * 32)

    p3 = _repack(o2, ho2 + 2, wp2, g2, 32, wp3)
    o3 = _stage_call(p3, _col_mask(ho3, wp3, wo3), h3, s3,
                     Wp=wp3, Ho=ho3, mode='down_pool', lanes=g3 * 64)

    pooled = o3.reshape(n, 64)
    return pooled @ fc_w + fc_b


# speedup vs baseline: 6.6860x; 1.5510x over previous
import jax
import jax.numpy as jnp
from jax.experimental import pallas as pl
from jax.experimental.pallas import tpu as pltpu


def _pade(y, a, b):
    num = a[5:6, :]
    for i in (4, 3, 2, 1, 0):
        num = num * y + a[i:i + 1, :]
    ay = jnp.abs(y)
    y2 = y * y
    den = 1.0 + ay * (b[0:1, :] + b[2:3, :] * y2) + y2 * (b[1:2, :] + b[3:4, :] * y2)
    return num * pl.reciprocal(den, approx=True)


def _make_stage(*, Wp, Ho, n_blocks, mode, out_kind, n_planes_rows=0, inv_hw=1.0):
    L = Ho * Wp
    OFF = Wp + 1
    CH = L
    for cand in (136, 144, 128, 96, 160):
        if L % cand == 0:
            CH = cand
            break
    if L <= 160:
        CH = L

    def body(*refs):
        it = iter(refs)
        x_ref = next(it)
        mask_ref = next(it)
        n_head = 4 if mode == 'stem' else 10
        head = [next(it) for _ in range(n_head)]
        blocks = [[next(it) for _ in range(8)] for _ in range(n_blocks)]
        sel_ref = next(it) if out_kind == 'planes' else None
        out_ref = next(it)
        cur = next(it)
        mid = next(it)

        Lp = cur.shape[0]
        lanes = cur.shape[1]
        cur[0:OFF, :] = jnp.zeros((OFF, lanes), jnp.float32)
        cur[OFF + L:Lp, :] = jnp.zeros((Lp - OFF - L, lanes), jnp.float32)
        mid[0:OFF, :] = jnp.zeros((OFF, lanes), jnp.float32)
        mid[OFF + L:Lp, :] = jnp.zeros((Lp - OFF - L, lanes), jnp.float32)

        def conv_act(tap, w_ref, sh, aa, bb, dst, extra=None):
            for q in range(0, L, CH):
                acc = None
                for dy in range(3):
                    for dx in range(3):
                        p = jnp.dot(tap(dy, dx, q), w_ref[dy * 3 + dx],
                                    preferred_element_type=jnp.float32)
                        acc = p if acc is None else acc + p
                y = acc + sh[...]
                if extra is not None:
                    y = y + extra(q)
                dst[OFF + q: OFF + q + CH, :] = (
                    _pade(y, aa, bb) * mask_ref[q: q + CH, :])

        def ctap(ref):
            return lambda dy, dx, q: ref[dy * Wp + dx + q: dy * Wp + dx + q + CH, :]

        if mode == 'stem':
            w, sh, a, b = head
            conv_act(lambda dy, dx, q: x_ref[0, dy * Wp + dx + q: dy * Wp + dx + q + CH, :],
                     w, sh, a, b, cur)
        else:
            w1, sh1, a1, b1, w2, sh2, a2, b2, ws, shs = head

            def ptap(dy, dx, q):
                off = (dy // 2) * Wp + (dx // 2) + q
                return x_ref[0, (dy % 2) * 2 + (dx % 2), off: off + CH, :]

            conv_act(ptap, w1, sh1, a1, b1, mid)

            def shortcut(q):
                return jnp.dot(x_ref[0, 3, q: q + CH, :], ws[...],
                               preferred_element_type=jnp.float32) + shs[...]

            conv_act(ctap(mid), w2, sh2, a2, b2, cur, extra=shortcut)

        for (w1, sh1, a1, b1, w2, sh2, a2, b2) in blocks:
            conv_act(ctap(cur), w1, sh1, a1, b1, mid)
            conv_act(ctap(mid), w2, sh2, a2, b2, cur,
                     extra=lambda q: cur[OFF + q: OFF + q + CH, :])

        if out_kind == 'pool':
            out_ref[0, 0, :] = jnp.sum(cur[...], axis=0) * inv_hw
        else:
            planes = jnp.dot(sel_ref[...], cur[...],
                             preferred_element_type=jnp.float32)
            out_ref[0, :, :, :] = planes.reshape(4, n_planes_rows, cur.shape[1])

    return body


def _full(arr):
    nd = arr.ndim
    return pl.BlockSpec(arr.shape, lambda n, _nd=nd: (0,) * _nd)


def _stage_call(x, x_spec, n_prog, mask, head, blocks, sel, *,
                Wp, Ho, mode, out_kind, lanes):
    Lp = (Ho + 2) * Wp + 8
    args = [x, mask] + list(head) + [a for blk in blocks for a in blk]
    if sel is not None:
        args.append(sel)
        n_planes_rows = sel.shape[0] // 4
    else:
        n_planes_rows = 0

    body = _make_stage(Wp=Wp, Ho=Ho, n_blocks=len(blocks), mode=mode,
                       out_kind=out_kind, n_planes_rows=n_planes_rows,
                       inv_hw=1.0 / (Ho * (Wp - 2)))

    in_specs = [x_spec] + [_full(a) for a in args[1:]]

    if out_kind == 'pool':
        out_shape = jax.ShapeDtypeStruct((n_prog, 1, lanes), jnp.float32)
        out_spec = pl.BlockSpec((1, 1, lanes), lambda n: (n, 0, 0))
    else:
        out_shape = jax.ShapeDtypeStruct((n_prog, 4, n_planes_rows, lanes), jnp.float32)
        out_spec = pl.BlockSpec((1, 4, n_planes_rows, lanes), lambda n: (n, 0, 0, 0))

    return pl.pallas_call(
        body,
        out_shape=out_shape,
        grid=(n_prog,),
        in_specs=in_specs,
        out_specs=out_spec,
        scratch_shapes=[pltpu.VMEM((Lp, lanes), jnp.float32),
                        pltpu.VMEM((Lp, lanes), jnp.float32)],
        compiler_params=pltpu.CompilerParams(dimension_semantics=("parallel",)),
    )(*args)


def _bd(w, g):
    eye = jnp.eye(g, dtype=w.dtype)
    if w.ndim == 2:
        ci, co = w.shape
        return jnp.einsum('ij,ab->iajb', eye, w).reshape(g * ci, g * co)
    t, ci, co = w.shape
    return jnp.einsum('ij,tab->tiajb', eye, w).reshape(t, g * ci, g * co)


def _tile(arr, g):
    return jnp.tile(arr, (1, g))


def _col_mask(ho, wp, wo):
    q = jnp.arange(ho * wp, dtype=jnp.int32)
    return ((q % wp) < wo).astype(jnp.float32)[:, None]


def _sel_matrix(hp, wp, wpn, lp):
    hq, wq = hp // 2, wp // 2
    rows = hq * wpn + 8
    py = jnp.arange(4)[:, None, None] // 2
    px = jnp.arange(4)[:, None, None] % 2
    i = jnp.arange(hq)[None, :, None]
    j = jnp.arange(wpn)[None, None, :]
    src = (2 * i + py) * wp + (2 * j + px)
    src = jnp.where(j < wq, src, -1)
    p = jax.nn.one_hot(src, lp, dtype=jnp.float32)
    p = p.reshape(4, hq * wpn, lp)
    p = jnp.pad(p, ((0, 0), (0, 8), (0, 0)))
    return p.reshape(4 * rows, lp)


def kernel(x, stem_w, stem_shift, stem_a, stem_b, s1b0_w1, s1b0_sh1, s1b0_a1, s1b0_b1, s1b0_w2, s1b0_sh2, s1b0_a2, s1b0_b2, s1b1_w1, s1b1_sh1, s1b1_a1, s1b1_b1, s1b1_w2, s1b1_sh2, s1b1_a2, s1b1_b2, s1b2_w1, s1b2_sh1, s1b2_a1, s1b2_b1, s1b2_w2, s1b2_sh2, s1b2_a2, s1b2_b2, s2h_w1, s2h_sh1, s2h_a1, s2h_b1, s2h_w2, s2h_sh2, s2h_a2, s2h_b2, s2h_ws, s2h_shs, s2b0_w1, s2b0_sh1, s2b0_a1, s2b0_b1, s2b0_w2, s2b0_sh2, s2b0_a2, s2b0_b2, s2b1_w1, s2b1_sh1, s2b1_a1, s2b1_b1, s2b1_w2, s2b1_sh2, s2b1_a2, s2b1_b2, s3h_w1, s3h_sh1, s3h_a1, s3h_b1, s3h_w2, s3h_sh2, s3h_a2, s3h_b2, s3h_ws, s3h_shs, s3b0_w1, s3b0_sh1, s3b0_a1, s3b0_b1, s3b0_w2, s3b0_sh2, s3b0_a2, s3b0_b2, s3b1_w1, s3b1_sh1, s3b1_a1, s3b1_b1, s3b1_w2, s3b1_sh2, s3b1_a2, s3b1_b2, fc_w, fc_b):
    n, cin, h, w = x.shape
    g1 = 16
    n1 = n // g1
    ho1, wo1 = h, w
    ho2, wo2 = h // 2, w // 2
    ho3, wo3 = h // 4, w // 4
    wp1, wp2, wp3 = wo1 + 2, wo2 + 2, wo3 + 2
    lp1 = (ho1 + 2) * wp1 + 8
    lp2 = (ho2 + 2) * wp2 + 8

    xt = jnp.transpose(x, (0, 2, 3, 1)).astype(jnp.float32)
    xp = jnp.pad(xt, ((0, 0), (1, 1), (1, 1), (0, 8 - cin)))
    xg = xp.reshape(n1, g1, (h + 2) * (w + 2), 8).transpose(0, 2, 1, 3)
    xg = jnp.pad(xg.reshape(n1, (h + 2) * (w + 2), g1 * 8), ((0, 0), (0, 8), (0, 0)))

    def pack_blk(blk, g):
        w1, sh1, a1, b1, w2, sh2, a2, b2 = blk
        return (_bd(w1, g), _tile(sh1, g), _tile(a1, g), _tile(b1, g),
                _bd(w2, g), _tile(sh2, g), _tile(a2, g), _tile(b2, g))

    stem = (_bd(stem_w, 16), _tile(stem_shift, 16), _tile(stem_a, 16), _tile(stem_b, 16))
    s1 = [pack_blk(b, 16) for b in
          ((s1b0_w1, s1b0_sh1, s1b0_a1, s1b0_b1, s1b0_w2, s1b0_sh2, s1b0_a2, s1b0_b2),
           (s1b1_w1, s1b1_sh1, s1b1_a1, s1b1_b1, s1b1_w2, s1b1_sh2, s1b1_a2, s1b1_b2),
           (s1b2_w1, s1b2_sh1, s1b2_a1, s1b2_b1, s1b2_w2, s1b2_sh2, s1b2_a2, s1b2_b2))]
    h2 = pack_blk((s2h_w1, s2h_sh1, s2h_a1, s2h_b1, s2h_w2, s2h_sh2, s2h_a2, s2h_b2), 8) \
        + (_bd(s2h_ws, 8), _tile(s2h_shs, 8))
    s2 = [pack_blk(b, 8) for b in
          ((s2b0_w1, s2b0_sh1, s2b0_a1, s2b0_b1, s2b0_w2, s2b0_sh2, s2b0_a2, s2b0_b2),
           (s2b1_w1, s2b1_sh1, s2b1_a1, s2b1_b1, s2b1_w2, s2b1_sh2, s2b1_a2, s2b1_b2))]
    h3 = pack_blk((s3h_w1, s3h_sh1, s3h_a1, s3h_b1, s3h_w2, s3h_sh2, s3h_a2, s3h_b2), 4) \
        + (_bd(s3h_ws, 4), _tile(s3h_shs, 4))
    s3 = [pack_blk(b, 4) for b in
          ((s3b0_w1, s3b0_sh1, s3b0_a1, s3b0_b1, s3b0_w2, s3b0_sh2, s3b0_a2, s3b0_b2),
           (s3b1_w1, s3b1_sh1, s3b1_a1, s3b1_b1, s3b1_w2, s3b1_sh2, s3b1_a2, s3b1_b2))]

    sel1 = _sel_matrix(ho1 + 2, wp1, wp2, lp1)
    sel2 = _sel_matrix(ho2 + 2, wp2, wp3, lp2)

    x_spec1 = pl.BlockSpec((1,) + tuple(xg.shape[1:]), lambda g: (g, 0, 0))
    p1 = _stage_call(xg, x_spec1, n1, _col_mask(ho1, wp1, wo1), stem, s1, sel1,
                     Wp=wp1, Ho=ho1, mode='stem', out_kind='planes', lanes=256)

    rows2 = sel1.shape[0] // 4
    x_spec2 = pl.BlockSpec((1, 4, rows2, 128), lambda g: (g // 2, 0, 0, g % 2))
    p2 = _stage_call(p1, x_spec2, 2 * n1, _col_mask(ho2, wp2, wo2), h2, s2, sel2,
                     Wp=wp2, Ho=ho2, mode='down', out_kind='planes', lanes=256)

    rows3 = sel2.shape[0] // 4
    x_spec3 = pl.BlockSpec((1, 4, rows3, 128), lambda g: (g // 2, 0, 0, g % 2))
    o3 = _stage_call(p2, x_spec3, 4 * n1, _col_mask(ho3, wp3, wo3), h3, s3, None,
                     Wp=wp3, Ho=ho3, mode='down', out_kind='pool', lanes=256)

    pooled = o3.reshape(n, 64)
    return pooled @ fc_w + fc_b


# speedup vs baseline: 6.9518x; 1.0398x over previous
import jax
import jax.numpy as jnp
from jax.experimental import pallas as pl
from jax.experimental.pallas import tpu as pltpu


def _pade(y, a, b):
    num = a[5:6, :]
    for i in (4, 3, 2, 1, 0):
        num = num * y + a[i:i + 1, :]
    ay = jnp.abs(y)
    y2 = y * y
    den = 1.0 + ay * (b[0:1, :] + b[2:3, :] * y2) + y2 * (b[1:2, :] + b[3:4, :] * y2)
    return num * pl.reciprocal(den, approx=True)


def _make_stage(*, Wp, Ho, n_blocks, mode, out_kind, n_planes_rows=0, inv_hw=1.0):
    L = Ho * Wp
    OFF = Wp + 1
    CH = L
    for cand in (272, 144, 128, 96, 160):
        if L % cand == 0:
            CH = cand
            break
    if L <= 160:
        CH = L

    def body(*refs):
        it = iter(refs)
        x_ref = next(it)
        mask_ref = next(it)
        n_head = 4 if mode == 'stem' else 10
        head = [next(it) for _ in range(n_head)]
        blocks = [[next(it) for _ in range(8)] for _ in range(n_blocks)]
        sel_ref = next(it) if out_kind == 'planes' else None
        out_ref = next(it)
        cur = next(it)
        mid = next(it)

        Lp = cur.shape[0]
        lanes = cur.shape[1]
        cur[0:OFF, :] = jnp.zeros((OFF, lanes), jnp.float32)
        cur[OFF + L:Lp, :] = jnp.zeros((Lp - OFF - L, lanes), jnp.float32)
        mid[0:OFF, :] = jnp.zeros((OFF, lanes), jnp.float32)
        mid[OFF + L:Lp, :] = jnp.zeros((Lp - OFF - L, lanes), jnp.float32)

        def conv_act(tap, w_ref, sh, aa, bb, dst, extra=None):
            for q in range(0, L, CH):
                acc = None
                for dy in range(3):
                    for dx in range(3):
                        p = jnp.dot(tap(dy, dx, q), w_ref[dy * 3 + dx],
                                    preferred_element_type=jnp.float32)
                        acc = p if acc is None else acc + p
                y = acc + sh[...]
                if extra is not None:
                    y = y + extra(q)
                dst[OFF + q: OFF + q + CH, :] = (
                    _pade(y, aa, bb) * mask_ref[q: q + CH, :])

        def ctap(ref):
            return lambda dy, dx, q: ref[dy * Wp + dx + q: dy * Wp + dx + q + CH, :]

        if mode == 'stem':
            w, sh, a, b = head
            conv_act(lambda dy, dx, q: x_ref[0, dy * Wp + dx + q: dy * Wp + dx + q + CH, :],
                     w, sh, a, b, cur)
        else:
            w1, sh1, a1, b1, w2, sh2, a2, b2, ws, shs = head

            def ptap(dy, dx, q):
                off = (dy // 2) * Wp + (dx // 2) + q
                return x_ref[0, (dy % 2) * 2 + (dx % 2), off: off + CH, :]

            conv_act(ptap, w1, sh1, a1, b1, mid)

            def shortcut(q):
                return jnp.dot(x_ref[0, 3, q: q + CH, :], ws[...],
                               preferred_element_type=jnp.float32) + shs[...]

            conv_act(ctap(mid), w2, sh2, a2, b2, cur, extra=shortcut)

        for (w1, sh1, a1, b1, w2, sh2, a2, b2) in blocks:
            conv_act(ctap(cur), w1, sh1, a1, b1, mid)
            conv_act(ctap(mid), w2, sh2, a2, b2, cur,
                     extra=lambda q: cur[OFF + q: OFF + q + CH, :])

        if out_kind == 'pool':
            out_ref[0, 0, :] = jnp.sum(cur[...], axis=0) * inv_hw
        else:
            planes = jnp.dot(sel_ref[...], cur[...],
                             preferred_element_type=jnp.float32)
            out_ref[0, :, :, :] = planes.reshape(
                4, n_planes_rows, cur.shape[1]).astype(jnp.bfloat16)

    return body


def _full(arr):
    nd = arr.ndim
    return pl.BlockSpec(arr.shape, lambda n, _nd=nd: (0,) * _nd)


def _stage_call(x, x_spec, n_prog, mask, head, blocks, sel, *,
                Wp, Ho, mode, out_kind, lanes):
    Lp = (Ho + 2) * Wp + 8
    args = [x, mask] + list(head) + [a for blk in blocks for a in blk]
    if sel is not None:
        args.append(sel)
        n_planes_rows = sel.shape[0] // 4
    else:
        n_planes_rows = 0

    body = _make_stage(Wp=Wp, Ho=Ho, n_blocks=len(blocks), mode=mode,
                       out_kind=out_kind, n_planes_rows=n_planes_rows,
                       inv_hw=1.0 / (Ho * (Wp - 2)))

    in_specs = [x_spec] + [_full(a) for a in args[1:]]

    if out_kind == 'pool':
        out_shape = jax.ShapeDtypeStruct((n_prog, 1, lanes), jnp.float32)
        out_spec = pl.BlockSpec((1, 1, lanes), lambda n: (n, 0, 0))
    else:
        out_shape = jax.ShapeDtypeStruct((n_prog, 4, n_planes_rows, lanes), jnp.bfloat16)
        out_spec = pl.BlockSpec((1, 4, n_planes_rows, lanes), lambda n: (n, 0, 0, 0))

    return pl.pallas_call(
        body,
        out_shape=out_shape,
        grid=(n_prog,),
        in_specs=in_specs,
        out_specs=out_spec,
        scratch_shapes=[pltpu.VMEM((Lp, lanes), jnp.float32),
                        pltpu.VMEM((Lp, lanes), jnp.float32)],
        compiler_params=pltpu.CompilerParams(dimension_semantics=("parallel",)),
    )(*args)


def _bd(w, g):
    eye = jnp.eye(g, dtype=w.dtype)
    if w.ndim == 2:
        ci, co = w.shape
        return jnp.einsum('ij,ab->iajb', eye, w).reshape(g * ci, g * co)
    t, ci, co = w.shape
    return jnp.einsum('ij,tab->tiajb', eye, w).reshape(t, g * ci, g * co)


def _tile(arr, g):
    return jnp.tile(arr, (1, g))


def _col_mask(ho, wp, wo):
    q = jnp.arange(ho * wp, dtype=jnp.int32)
    return ((q % wp) < wo).astype(jnp.float32)[:, None]


def _sel_matrix(hp, wp, wpn, lp):
    hq, wq = hp // 2, wp // 2
    rows = hq * wpn + 8
    py = jnp.arange(4)[:, None, None] // 2
    px = jnp.arange(4)[:, None, None] % 2
    i = jnp.arange(hq)[None, :, None]
    j = jnp.arange(wpn)[None, None, :]
    src = (2 * i + py) * wp + (2 * j + px)
    src = jnp.where(j < wq, src, -1)
    p = jax.nn.one_hot(src, lp, dtype=jnp.float32)
    p = p.reshape(4, hq * wpn, lp)
    p = jnp.pad(p, ((0, 0), (0, 8), (0, 0)))
    return p.reshape(4 * rows, lp)


def kernel(x, stem_w, stem_shift, stem_a, stem_b, s1b0_w1, s1b0_sh1, s1b0_a1, s1b0_b1, s1b0_w2, s1b0_sh2, s1b0_a2, s1b0_b2, s1b1_w1, s1b1_sh1, s1b1_a1, s1b1_b1, s1b1_w2, s1b1_sh2, s1b1_a2, s1b1_b2, s1b2_w1, s1b2_sh1, s1b2_a1, s1b2_b1, s1b2_w2, s1b2_sh2, s1b2_a2, s1b2_b2, s2h_w1, s2h_sh1, s2h_a1, s2h_b1, s2h_w2, s2h_sh2, s2h_a2, s2h_b2, s2h_ws, s2h_shs, s2b0_w1, s2b0_sh1, s2b0_a1, s2b0_b1, s2b0_w2, s2b0_sh2, s2b0_a2, s2b0_b2, s2b1_w1, s2b1_sh1, s2b1_a1, s2b1_b1, s2b1_w2, s2b1_sh2, s2b1_a2, s2b1_b2, s3h_w1, s3h_sh1, s3h_a1, s3h_b1, s3h_w2, s3h_sh2, s3h_a2, s3h_b2, s3h_ws, s3h_shs, s3b0_w1, s3b0_sh1, s3b0_a1, s3b0_b1, s3b0_w2, s3b0_sh2, s3b0_a2, s3b0_b2, s3b1_w1, s3b1_sh1, s3b1_a1, s3b1_b1, s3b1_w2, s3b1_sh2, s3b1_a2, s3b1_b2, fc_w, fc_b):
    n, cin, h, w = x.shape
    g1 = 16
    n1 = n // g1
    ho1, wo1 = h, w
    ho2, wo2 = h // 2, w // 2
    ho3, wo3 = h // 4, w // 4
    wp1, wp2, wp3 = wo1 + 2, wo2 + 2, wo3 + 2
    lp1 = (ho1 + 2) * wp1 + 8
    lp2 = (ho2 + 2) * wp2 + 8

    xt = jnp.transpose(x, (0, 2, 3, 1)).astype(jnp.float32)
    xp = jnp.pad(xt, ((0, 0), (1, 1), (1, 1), (0, 8 - cin)))
    xg = xp.reshape(n1, g1, (h + 2) * (w + 2), 8).transpose(0, 2, 1, 3)
    xg = jnp.pad(xg.reshape(n1, (h + 2) * (w + 2), g1 * 8), ((0, 0), (0, 8), (0, 0)))
    xg = xg.astype(jnp.bfloat16)

    def pack_blk(blk, g):
        w1, sh1, a1, b1, w2, sh2, a2, b2 = blk
        return (_bd(w1, g), _tile(sh1, g), _tile(a1, g), _tile(b1, g),
                _bd(w2, g), _tile(sh2, g), _tile(a2, g), _tile(b2, g))

    def bf(t, idxs):
        return tuple(a.astype(jnp.bfloat16) if i in idxs else a
                     for i, a in enumerate(t))

    stem = bf((_bd(stem_w, 16), _tile(stem_shift, 16),
               _tile(stem_a, 16), _tile(stem_b, 16)), (0,))
    s1 = [pack_blk(b, 16) for b in
          ((s1b0_w1, s1b0_sh1, s1b0_a1, s1b0_b1, s1b0_w2, s1b0_sh2, s1b0_a2, s1b0_b2),
           (s1b1_w1, s1b1_sh1, s1b1_a1, s1b1_b1, s1b1_w2, s1b1_sh2, s1b1_a2, s1b1_b2),
           (s1b2_w1, s1b2_sh1, s1b2_a1, s1b2_b1, s1b2_w2, s1b2_sh2, s1b2_a2, s1b2_b2))]
    h2 = bf(pack_blk((s2h_w1, s2h_sh1, s2h_a1, s2h_b1, s2h_w2, s2h_sh2, s2h_a2, s2h_b2), 8)
            + (_bd(s2h_ws, 8), _tile(s2h_shs, 8)), (0, 8))
    s2 = [pack_blk(b, 8) for b in
          ((s2b0_w1, s2b0_sh1, s2b0_a1, s2b0_b1, s2b0_w2, s2b0_sh2, s2b0_a2, s2b0_b2),
           (s2b1_w1, s2b1_sh1, s2b1_a1, s2b1_b1, s2b1_w2, s2b1_sh2, s2b1_a2, s2b1_b2))]
    h3 = bf(pack_blk((s3h_w1, s3h_sh1, s3h_a1, s3h_b1, s3h_w2, s3h_sh2, s3h_a2, s3h_b2), 4)
            + (_bd(s3h_ws, 4), _tile(s3h_shs, 4)), (0, 8))
    s3 = [pack_blk(b, 4) for b in
          ((s3b0_w1, s3b0_sh1, s3b0_a1, s3b0_b1, s3b0_w2, s3b0_sh2, s3b0_a2, s3b0_b2),
           (s3b1_w1, s3b1_sh1, s3b1_a1, s3b1_b1, s3b1_w2, s3b1_sh2, s3b1_a2, s3b1_b2))]

    sel1 = _sel_matrix(ho1 + 2, wp1, wp2, lp1)
    sel2 = _sel_matrix(ho2 + 2, wp2, wp3, lp2)

    x_spec1 = pl.BlockSpec((1,) + tuple(xg.shape[1:]), lambda g: (g, 0, 0))
    p1 = _stage_call(xg, x_spec1, n1, _col_mask(ho1, wp1, wo1), stem, s1, sel1,
                     Wp=wp1, Ho=ho1, mode='stem', out_kind='planes', lanes=256)

    rows2 = sel1.shape[0] // 4
    x_spec2 = pl.BlockSpec((1, 4, rows2, 128), lambda g: (g // 2, 0, 0, g % 2))
    p2 = _stage_call(p1, x_spec2, 2 * n1, _col_mask(ho2, wp2, wo2), h2, s2, sel2,
                     Wp=wp2, Ho=ho2, mode='down', out_kind='planes', lanes=256)

    rows3 = sel2.shape[0] // 4
    x_spec3 = pl.BlockSpec((1, 4, rows3, 128), lambda g: (g // 2, 0, 0, g % 2))
    o3 = _stage_call(p2, x_spec3, 4 * n1, _col_mask(ho3, wp3, wo3), h3, s3, None,
                     Wp=wp3, Ho=ho3, mode='down', out_kind='pool', lanes=256)

    pooled = o3.reshape(n, 64)
    return pooled @ fc_w + fc_b


# speedup vs baseline: 7.9684x; 1.1462x over previous
import jax
import jax.numpy as jnp
from jax.experimental import pallas as pl
from jax.experimental.pallas import tpu as pltpu


def _pade(y, a, b):
    num = a[5:6, :]
    for i in (4, 3, 2, 1, 0):
        num = num * y + a[i:i + 1, :]
    ay = jnp.abs(y)
    y2 = y * y
    den = 1.0 + ay * (b[0:1, :] + b[2:3, :] * y2) + y2 * (b[1:2, :] + b[3:4, :] * y2)
    return num * pl.reciprocal(den, approx=True)


def _make_stage(*, Wp, Ho, n_blocks, mode, out_kind, n_planes_rows=0, inv_hw=1.0):
    L = Ho * Wp
    OFF = Wp + 1
    CH = L
    for cand in (272, 144, 128, 96, 160):
        if L % cand == 0:
            CH = cand
            break
    if L <= 160:
        CH = L

    def body(*refs):
        it = iter(refs)
        x_ref = next(it)
        mask_ref = next(it)
        n_head = 4 if mode == 'stem' else 10
        head = [next(it) for _ in range(n_head)]
        blocks = [[next(it) for _ in range(8)] for _ in range(n_blocks)]
        sel_ref = next(it) if out_kind == 'planes' else None
        out_ref = next(it)
        cur = next(it)
        mid = next(it)

        Lp = cur.shape[0]
        lanes = cur.shape[1]
        cur[0:OFF, :] = jnp.zeros((OFF, lanes), jnp.float32)
        cur[OFF + L:Lp, :] = jnp.zeros((Lp - OFF - L, lanes), jnp.float32)
        mid[0:OFF, :] = jnp.zeros((OFF, lanes), jnp.float32)
        mid[OFF + L:Lp, :] = jnp.zeros((Lp - OFF - L, lanes), jnp.float32)

        def conv_act(tap, w_ref, sh, aa, bb, dst, extra=None):
            for q in range(0, L, CH):
                acc = None
                for dy in range(3):
                    for dx in range(3):
                        p = jnp.dot(tap(dy, dx, q), w_ref[dy * 3 + dx],
                                    preferred_element_type=jnp.float32)
                        acc = p if acc is None else acc + p
                y = acc + sh[...]
                if extra is not None:
                    y = y + extra(q)
                dst[OFF + q: OFF + q + CH, :] = (
                    _pade(y, aa, bb) * mask_ref[q: q + CH, :])

        def ctap(ref):
            return lambda dy, dx, q: ref[dy * Wp + dx + q: dy * Wp + dx + q + CH, :]

        if mode == 'stem':
            w, sh, a, b = head
            conv_act(lambda dy, dx, q: x_ref[0, dy * Wp + dx + q: dy * Wp + dx + q + CH, :],
                     w, sh, a, b, cur)
        else:
            w1, sh1, a1, b1, w2, sh2, a2, b2, ws, shs = head

            def ptap(dy, dx, q):
                off = (dy // 2) * Wp + (dx // 2) + q
                return x_ref[0, (dy % 2) * 2 + (dx % 2), off: off + CH, :]

            conv_act(ptap, w1, sh1, a1, b1, mid)

            def shortcut(q):
                return jnp.dot(x_ref[0, 3, q: q + CH, :], ws[...],
                               preferred_element_type=jnp.float32) + shs[...]

            conv_act(ctap(mid), w2, sh2, a2, b2, cur, extra=shortcut)

        for (w1, sh1, a1, b1, w2, sh2, a2, b2) in blocks:
            conv_act(ctap(cur), w1, sh1, a1, b1, mid)
            conv_act(ctap(mid), w2, sh2, a2, b2, cur,
                     extra=lambda q: cur[OFF + q: OFF + q + CH, :])

        if out_kind == 'pool':
            out_ref[0, 0, :] = jnp.sum(cur[...], axis=0) * inv_hw
        else:
            planes = jnp.dot(sel_ref[...], cur[...],
                             preferred_element_type=jnp.float32)
            out_ref[0, :, :, :] = planes.reshape(
                4, n_planes_rows, cur.shape[1]).astype(jnp.bfloat16)

    return body


def _make_s3_stacked(*, Wp, Ho, n_blocks):
    OFF = Wp + 1
    L = Ho * Wp
    GS = (Ho + 2) * Wp + 8
    SL = 4
    TM = SL * GS
    PB = 16
    CH = 144 if TM % 144 == 0 else TM
    inv = 1.0 / (Ho * (Wp - 2))

    def body(*refs):
        it = iter(refs)
        x0_ref = next(it)
        x1_ref = next(it)
        mask_ref = next(it)
        head = [next(it) for _ in range(10)]
        blocks = [[next(it) for _ in range(8)] for _ in range(n_blocks)]
        out_ref = next(it)
        cur = next(it)
        mid = next(it)
        psc = next(it)

        w1, sh1, a1, b1, w2, sh2, a2, b2, ws, shs = head

        psc[...] = jnp.zeros_like(psc)
        pr = x0_ref.shape[2]
        for pl in range(4):
            for si, src in enumerate((x0_ref, x1_ref)):
                for g in range(2):
                    k = 2 * si + g
                    psc[pl, PB + k * GS: PB + k * GS + pr, :] = \
                        src[0, pl, :, g * 128:(g + 1) * 128]

        lanes = cur.shape[1]
        cur[0:PB, :] = jnp.zeros((PB, lanes), jnp.float32)
        cur[PB + TM:, :] = jnp.zeros((cur.shape[0] - PB - TM, lanes), jnp.float32)
        mid[0:PB, :] = jnp.zeros((PB, lanes), jnp.float32)
        mid[PB + TM:, :] = jnp.zeros((mid.shape[0] - PB - TM, lanes), jnp.float32)

        def conv_act(tap, w_ref, sh, aa, bb, dst, extra=None):
            for q in range(0, TM, CH):
                acc = None
                for dy in range(3):
                    for dx in range(3):
                        p = jnp.dot(tap(dy, dx, q), w_ref[dy * 3 + dx],
                                    preferred_element_type=jnp.float32)
                        acc = p if acc is None else acc + p
                y = acc + sh[...]
                if extra is not None:
                    y = y + extra(q)
                dst[PB + q: PB + q + CH, :] = (
                    _pade(y, aa, bb) * mask_ref[q: q + CH, :])

        def ptap(dy, dx, q):
            base = PB - OFF + (dy // 2) * Wp + (dx // 2) + q
            return psc[(dy % 2) * 2 + (dx % 2), base: base + CH, :]

        def ctap(ref):
            def t(dy, dx, q):
                base = PB - OFF + dy * Wp + dx + q
                return ref[base: base + CH, :]
            return t

        conv_act(ptap, w1, sh1, a1, b1, mid)

        def shortcut(q):
            return jnp.dot(psc[3, PB - OFF + q: PB - OFF + q + CH, :], ws[...],
                           preferred_element_type=jnp.float32) + shs[...]

        conv_act(ctap(mid), w2, sh2, a2, b2, cur, extra=shortcut)

        for (bw1, bsh1, ba1, bb1, bw2, bsh2, ba2, bb2) in blocks:
            conv_act(ctap(cur), bw1, bsh1, ba1, bb1, mid)
            conv_act(ctap(mid), bw2, bsh2, ba2, bb2, cur,
                     extra=lambda q: cur[PB + q: PB + q + CH, :])

        for k in range(SL):
            out_ref[0, k, :] = jnp.sum(cur[PB + k * GS: PB + (k + 1) * GS, :],
                                       axis=0) * inv

    return body, TM, PB


def _stacked_mask(ho, wp, wo, gs, sl):
    r = jnp.arange(sl * gs, dtype=jnp.int32)
    t = r % gs - (wp + 1)
    valid = (t >= 0) & (t < ho * wp) & ((t % wp) < wo)
    return valid.astype(jnp.float32)[:, None]


def _stage3_stacked_call(p2, mask3, head, blocks, *, Wp, Ho, lanes):
    n3 = p2.shape[0] // 2
    rows = p2.shape[2]
    body, TM, PB = _make_s3_stacked(Wp=Wp, Ho=Ho, n_blocks=len(blocks))
    args = [p2, p2, mask3] + list(head) + [a for blk in blocks for a in blk]

    in_specs = [pl.BlockSpec((1, 4, rows, 256), lambda n: (2 * n, 0, 0, 0)),
                pl.BlockSpec((1, 4, rows, 256), lambda n: (2 * n + 1, 0, 0, 0))]
    in_specs += [_full(a) for a in args[2:]]

    return pl.pallas_call(
        body,
        out_shape=jax.ShapeDtypeStruct((n3, 4, lanes), jnp.float32),
        grid=(n3,),
        in_specs=in_specs,
        out_specs=pl.BlockSpec((1, 4, lanes), lambda n: (n, 0, 0)),
        scratch_shapes=[pltpu.VMEM((PB + TM + 16, lanes), jnp.float32),
                        pltpu.VMEM((PB + TM + 16, lanes), jnp.float32),
                        pltpu.VMEM((4, PB + TM, 128), jnp.bfloat16)],
        compiler_params=pltpu.CompilerParams(dimension_semantics=("parallel",)),
    )(*args)


def _full(arr):
    nd = arr.ndim
    return pl.BlockSpec(arr.shape, lambda n, _nd=nd: (0,) * _nd)


def _stage_call(x, x_spec, n_prog, mask, head, blocks, sel, *,
                Wp, Ho, mode, out_kind, lanes):
    Lp = (Ho + 2) * Wp + 8
    args = [x, mask] + list(head) + [a for blk in blocks for a in blk]
    if sel is not None:
        args.append(sel)
        n_planes_rows = sel.shape[0] // 4
    else:
        n_planes_rows = 0

    body = _make_stage(Wp=Wp, Ho=Ho, n_blocks=len(blocks), mode=mode,
                       out_kind=out_kind, n_planes_rows=n_planes_rows,
                       inv_hw=1.0 / (Ho * (Wp - 2)))

    in_specs = [x_spec] + [_full(a) for a in args[1:]]

    if out_kind == 'pool':
        out_shape = jax.ShapeDtypeStruct((n_prog, 1, lanes), jnp.float32)
        out_spec = pl.BlockSpec((1, 1, lanes), lambda n: (n, 0, 0))
    else:
        out_shape = jax.ShapeDtypeStruct((n_prog, 4, n_planes_rows, lanes), jnp.bfloat16)
        out_spec = pl.BlockSpec((1, 4, n_planes_rows, lanes), lambda n: (n, 0, 0, 0))

    return pl.pallas_call(
        body,
        out_shape=out_shape,
        grid=(n_prog,),
        in_specs=in_specs,
        out_specs=out_spec,
        scratch_shapes=[pltpu.VMEM((Lp, lanes), jnp.float32),
                        pltpu.VMEM((Lp, lanes), jnp.float32)],
        compiler_params=pltpu.CompilerParams(dimension_semantics=("parallel",)),
    )(*args)


def _bd(w, g):
    eye = jnp.eye(g, dtype=w.dtype)
    if w.ndim == 2:
        ci, co = w.shape
        return jnp.einsum('ij,ab->iajb', eye, w).reshape(g * ci, g * co)
    t, ci, co = w.shape
    return jnp.einsum('ij,tab->tiajb', eye, w).reshape(t, g * ci, g * co)


def _tile(arr, g):
    return jnp.tile(arr, (1, g))


def _col_mask(ho, wp, wo):
    q = jnp.arange(ho * wp, dtype=jnp.int32)
    return ((q % wp) < wo).astype(jnp.float32)[:, None]


def _sel_matrix(hp, wp, wpn, lp):
    hq, wq = hp // 2, wp // 2
    rows = hq * wpn + 8
    py = jnp.arange(4)[:, None, None] // 2
    px = jnp.arange(4)[:, None, None] % 2
    i = jnp.arange(hq)[None, :, None]
    j = jnp.arange(wpn)[None, None, :]
    src = (2 * i + py) * wp + (2 * j + px)
    src = jnp.where(j < wq, src, -1)
    p = jax.nn.one_hot(src, lp, dtype=jnp.float32)
    p = p.reshape(4, hq * wpn, lp)
    p = jnp.pad(p, ((0, 0), (0, 8), (0, 0)))
    return p.reshape(4 * rows, lp)


def kernel(x, stem_w, stem_shift, stem_a, stem_b, s1b0_w1, s1b0_sh1, s1b0_a1, s1b0_b1, s1b0_w2, s1b0_sh2, s1b0_a2, s1b0_b2, s1b1_w1, s1b1_sh1, s1b1_a1, s1b1_b1, s1b1_w2, s1b1_sh2, s1b1_a2, s1b1_b2, s1b2_w1, s1b2_sh1, s1b2_a1, s1b2_b1, s1b2_w2, s1b2_sh2, s1b2_a2, s1b2_b2, s2h_w1, s2h_sh1, s2h_a1, s2h_b1, s2h_w2, s2h_sh2, s2h_a2, s2h_b2, s2h_ws, s2h_shs, s2b0_w1, s2b0_sh1, s2b0_a1, s2b0_b1, s2b0_w2, s2b0_sh2, s2b0_a2, s2b0_b2, s2b1_w1, s2b1_sh1, s2b1_a1, s2b1_b1, s2b1_w2, s2b1_sh2, s2b1_a2, s2b1_b2, s3h_w1, s3h_sh1, s3h_a1, s3h_b1, s3h_w2, s3h_sh2, s3h_a2, s3h_b2, s3h_ws, s3h_shs, s3b0_w1, s3b0_sh1, s3b0_a1, s3b0_b1, s3b0_w2, s3b0_sh2, s3b0_a2, s3b0_b2, s3b1_w1, s3b1_sh1, s3b1_a1, s3b1_b1, s3b1_w2, s3b1_sh2, s3b1_a2, s3b1_b2, fc_w, fc_b):
    n, cin, h, w = x.shape
    g1 = 16
    n1 = n // g1
    ho1, wo1 = h, w
    ho2, wo2 = h // 2, w // 2
    ho3, wo3 = h // 4, w // 4
    wp1, wp2, wp3 = wo1 + 2, wo2 + 2, wo3 + 2
    lp1 = (ho1 + 2) * wp1 + 8
    lp2 = (ho2 + 2) * wp2 + 8

    xt = jnp.transpose(x, (0, 2, 3, 1)).astype(jnp.float32)
    xp = jnp.pad(xt, ((0, 0), (1, 1), (1, 1), (0, 8 - cin)))
    xg = xp.reshape(n1, g1, (h + 2) * (w + 2), 8).transpose(0, 2, 1, 3)
    xg = jnp.pad(xg.reshape(n1, (h + 2) * (w + 2), g1 * 8), ((0, 0), (0, 8), (0, 0)))
    xg = xg.astype(jnp.bfloat16)

    def pack_blk(blk, g):
        w1, sh1, a1, b1, w2, sh2, a2, b2 = blk
        return (_bd(w1, g), _tile(sh1, g), _tile(a1, g), _tile(b1, g),
                _bd(w2, g), _tile(sh2, g), _tile(a2, g), _tile(b2, g))

    def bf(t, idxs):
        return tuple(a.astype(jnp.bfloat16) if i in idxs else a
                     for i, a in enumerate(t))

    stem = bf((_bd(stem_w, 16), _tile(stem_shift, 16),
               _tile(stem_a, 16), _tile(stem_b, 16)), (0,))
    s1 = [pack_blk(b, 16) for b in
          ((s1b0_w1, s1b0_sh1, s1b0_a1, s1b0_b1, s1b0_w2, s1b0_sh2, s1b0_a2, s1b0_b2),
           (s1b1_w1, s1b1_sh1, s1b1_a1, s1b1_b1, s1b1_w2, s1b1_sh2, s1b1_a2, s1b1_b2),
           (s1b2_w1, s1b2_sh1, s1b2_a1, s1b2_b1, s1b2_w2, s1b2_sh2, s1b2_a2, s1b2_b2))]
    h2 = bf(pack_blk((s2h_w1, s2h_sh1, s2h_a1, s2h_b1, s2h_w2, s2h_sh2, s2h_a2, s2h_b2), 8)
            + (_bd(s2h_ws, 8), _tile(s2h_shs, 8)), (0, 8))
    s2 = [pack_blk(b, 8) for b in
          ((s2b0_w1, s2b0_sh1, s2b0_a1, s2b0_b1, s2b0_w2, s2b0_sh2, s2b0_a2, s2b0_b2),
           (s2b1_w1, s2b1_sh1, s2b1_a1, s2b1_b1, s2b1_w2, s2b1_sh2, s2b1_a2, s2b1_b2))]
    h3 = bf(pack_blk((s3h_w1, s3h_sh1, s3h_a1, s3h_b1, s3h_w2, s3h_sh2, s3h_a2, s3h_b2), 4)
            + (_bd(s3h_ws, 4), _tile(s3h_shs, 4)), (0, 8))
    s3 = [pack_blk(b, 4) for b in
          ((s3b0_w1, s3b0_sh1, s3b0_a1, s3b0_b1, s3b0_w2, s3b0_sh2, s3b0_a2, s3b0_b2),
           (s3b1_w1, s3b1_sh1, s3b1_a1, s3b1_b1, s3b1_w2, s3b1_sh2, s3b1_a2, s3b1_b2))]

    sel1 = _sel_matrix(ho1 + 2, wp1, wp2, lp1)
    sel2 = _sel_matrix(ho2 + 2, wp2, wp3, lp2)

    x_spec1 = pl.BlockSpec((1,) + tuple(xg.shape[1:]), lambda g: (g, 0, 0))
    p1 = _stage_call(xg, x_spec1, n1, _col_mask(ho1, wp1, wo1), stem, s1, sel1,
                     Wp=wp1, Ho=ho1, mode='stem', out_kind='planes', lanes=256)

    rows2 = sel1.shape[0] // 4
    x_spec2 = pl.BlockSpec((1, 4, rows2, 128), lambda g: (g // 2, 0, 0, g % 2))
    p2 = _stage_call(p1, x_spec2, 2 * n1, _col_mask(ho2, wp2, wo2), h2, s2, sel2,
                     Wp=wp2, Ho=ho2, mode='down', out_kind='planes', lanes=256)

    gs3 = (ho3 + 2) * wp3 + 8
    o3 = _stage3_stacked_call(p2, _stacked_mask(ho3, wp3, wo3, gs3, 4), h3, s3,
                              Wp=wp3, Ho=ho3, lanes=256)

    pooled = o3.reshape(n, 64)
    return pooled @ fc_w + fc_b
